# baseline (speedup 1.0000x reference)
import functools

import jax
import jax.numpy as jnp
from jax import lax
from jax.experimental import pallas as pl
from jax.experimental.pallas import tpu as pltpu
from jax.experimental.pallas import tpu_sc as plsc

N_NODES = 10000
N_EDGES = 160000
N_PAIRS = 170000
F_OUT = 128

CH = 10752
PPAD = 16 * CH
NJ = CH // 128
NV = CH // 16
HALF = 50_000_000
DUMMY = 12_400_000
TSIZE = 12_500_008
SROWS = 10240


def _sc_body(nodes_hbm, src_hbm, dst_hbm, wa_hbm,
             s32_hbm, ps_hbm,
             t_hbm, nodes_v, src_v, dst_v, key2_v, pval_v, e_v, t_v, s_local,
             wa_v, row_v, sem):
    c = lax.axis_index("c")
    s = lax.axis_index("s")
    base = s * CH

    pltpu.sync_copy(nodes_hbm, nodes_v)
    pltpu.sync_copy(wa_hbm, wa_v)
    pltpu.sync_copy(src_hbm.at[pl.ds(base, CH)], src_v)
    pltpu.sync_copy(dst_hbm.at[pl.ds(base, CH)], dst_v)

    def dot_body(i, carry):
        a1v, a2v = carry
        w = wa_v[pl.ds(i * 16, 16)]
        return (a1v + w * wa_v[pl.ds(128 + i * 16, 16)],
                a2v + w * wa_v[pl.ds(256 + i * 16, 16)])
    zero16 = jnp.zeros((16,), jnp.float32)
    acc1, acc2 = lax.fori_loop(0, 8, dot_body, (zero16, zero16))
    c1 = jnp.sum(acc1, axis=0)
    c2 = jnp.sum(acc2, axis=0)

    lanes = lax.iota(jnp.int32, 16)
    kdummy = DUMMY + c
    klo = c * HALF
    khi = klo + HALF

    def e_body(i, acc):
        sv = src_v[pl.ds(i * 16, 16)]
        dv = dst_v[pl.ds(i * 16, 16)]
        pv = base + i * 16 + lanes
        ns = plsc.load_gather(nodes_v, [sv])
        nd = plsc.load_gather(nodes_v, [dv])
        z = c1 * ns + c2 * nd
        z = jnp.maximum(z, z * jnp.float32(0.01))
        valid = pv < N_PAIRS
        e = jnp.where(valid, jnp.exp(z), jnp.float32(0.0))
        e_v[pl.ds(i * 16, 16)] = e
        key = sv * N_NODES + dv
        own = valid & (key >= klo) & (key < khi)
        key = key % 12_000_000
        key2_v[pl.ds(i * 16, 16)] = jnp.where(own, key, kdummy)
        pval_v[pl.ds(i * 16, 16)] = pv
        return acc + e
    acc_e = lax.fori_loop(0, NV, e_body, zero16)

    row_v[pl.ds(0, 16)] = acc_e
    pltpu.sync_copy(row_v, ps_hbm.at[c * 16 + s])

    pltpu.async_copy(pval_v, t_hbm.at[key2_v], sem).wait()

    plsc.subcore_barrier()

    pltpu.async_copy(t_hbm.at[key2_v], t_v, sem).wait()

    def zero_body(k, _):
        s_local[pl.ds(k * 16, 16)] = zero16
        return 0
    lax.fori_loop(0, SROWS // 16, zero_body, 0)

    def acc_body(i, _):
        kv = key2_v[pl.ds(i * 16, 16)]
        pv = pval_v[pl.ds(i * 16, 16)]
        tv = t_v[pl.ds(i * 16, 16)]
        m = (kv != kdummy) & (tv == pv)
        dv = dst_v[pl.ds(i * 16, 16)]
        sv = src_v[pl.ds(i * 16, 16)]
        w = e_v[pl.ds(i * 16, 16)] * plsc.load_gather(nodes_v, [dv])
        plsc.addupdate_scatter(s_local, [sv], w, mask=m)
        return 0
    lax.fori_loop(0, NV, acc_body, 0)

    pltpu.sync_copy(s_local, s32_hbm.at[c * 16 + s])


def _tc_body(s32_ref, ps_ref, w_ref, o_ref):
    denom = jnp.sum(ps_ref[:16, :])
    ssum = jnp.sum(s32_ref[...], axis=0, keepdims=True)
    ssum = ssum[:, :N_NODES] * (jnp.float32(1.0) / denom)
    out = lax.dot_general(ssum, w_ref[...], (((0,), (0,)), ((), ())),
                          preferred_element_type=jnp.float32)
    o_ref[0] = jnp.where(out > 0, out, out * jnp.float32(0.01))


@jax.jit
def kernel(x, src, dst, W_node, a):
    nodes = x[0, N_EDGES:]
    srcp = jnp.pad(src.astype(jnp.int32), (0, PPAD - N_PAIRS))
    dstp = jnp.pad(dst.astype(jnp.int32), (0, PPAD - N_PAIRS))
    wa = jnp.concatenate([W_node[0], a[:F_OUT, 0], a[F_OUT:, 0]])

    mesh = plsc.VectorSubcoreMesh(core_axis_name="c", subcore_axis_name="s",
                                  num_cores=2, num_subcores=16)
    sc = pl.kernel(
        _sc_body,
        mesh=mesh,
        compiler_params=pltpu.CompilerParams(needs_layout_passes=False),
        out_type=[
            jax.ShapeDtypeStruct((32, SROWS), jnp.float32),
            jax.ShapeDtypeStruct((32, 16), jnp.float32),
        ],
        scratch_types=[
            pltpu.HBM((TSIZE,), jnp.int32),
            pltpu.VMEM((N_NODES,), jnp.float32),
            pltpu.VMEM((CH,), jnp.int32),
            pltpu.VMEM((CH,), jnp.int32),
            pltpu.VMEM((CH,), jnp.int32),
            pltpu.VMEM((CH,), jnp.int32),
            pltpu.VMEM((CH,), jnp.float32),
            pltpu.VMEM((CH,), jnp.int32),
            pltpu.VMEM((SROWS,), jnp.float32),
            pltpu.VMEM((384,), jnp.float32),
            pltpu.VMEM((16,), jnp.float32),
            pltpu.SemaphoreType.DMA,
        ],
    )
    s32, ps = sc(nodes, srcp, dstp, wa)

    out = pl.pallas_call(
        _tc_body,
        out_shape=jax.ShapeDtypeStruct((1, N_NODES, F_OUT), jnp.float32),
    )(s32, ps, W_node)
    return out


# speedup vs baseline: 1.0108x; 1.0108x over previous
import functools

import jax
import jax.numpy as jnp
from jax import lax
from jax.experimental import pallas as pl
from jax.experimental.pallas import tpu as pltpu
from jax.experimental.pallas import tpu_sc as plsc

N_NODES = 10000
N_EDGES = 160000
N_PAIRS = 170000
F_OUT = 128

CH = 10752
PPAD = 16 * CH
NJ = CH // 128
NV = CH // 16
HALF = 50_000_000
DUMMY = 100_000_000
TSIZE = 100_000_008
SROWS = 10240


def _sc_body(nodes_hbm, src_hbm, dst_hbm, wa_hbm,
             s32_hbm, ps_hbm,
             t_hbm, nodes_v, src_v, dst_v, key2_v, pval_v, e_v, t_v, s_local,
             wa_v, row_v, sem):
    c = lax.axis_index("c")
    s = lax.axis_index("s")
    base = s * CH

    with jax.named_scope("ph0_stage"):
        pltpu.sync_copy(nodes_hbm, nodes_v)
        pltpu.sync_copy(wa_hbm, wa_v)
        pltpu.sync_copy(src_hbm.at[pl.ds(base, CH)], src_v)
        pltpu.sync_copy(dst_hbm.at[pl.ds(base, CH)], dst_v)

    def dot_body(i, carry):
        a1v, a2v = carry
        w = wa_v[pl.ds(i * 16, 16)]
        return (a1v + w * wa_v[pl.ds(128 + i * 16, 16)],
                a2v + w * wa_v[pl.ds(256 + i * 16, 16)])
    zero16 = jnp.zeros((16,), jnp.float32)
    acc1, acc2 = lax.fori_loop(0, 8, dot_body, (zero16, zero16))
    c1 = jnp.sum(acc1, axis=0)
    c2 = jnp.sum(acc2, axis=0)

    lanes = lax.iota(jnp.int32, 16)
    kdummy = DUMMY + c
    klo = c * HALF
    khi = klo + HALF

    def e_body(i, acc):
        sv = src_v[pl.ds(i * 16, 16)]
        dv = dst_v[pl.ds(i * 16, 16)]
        pv = base + i * 16 + lanes
        ns = plsc.load_gather(nodes_v, [sv])
        nd = plsc.load_gather(nodes_v, [dv])
        z = c1 * ns + c2 * nd
        z = jnp.maximum(z, z * jnp.float32(0.01))
        valid = pv < N_PAIRS
        e = jnp.where(valid, jnp.exp(z), jnp.float32(0.0))
        e_v[pl.ds(i * 16, 16)] = e
        key = sv * N_NODES + dv
        own = valid & (key >= klo) & (key < khi)
        key2_v[pl.ds(i * 16, 16)] = jnp.where(own, key, kdummy)
        pval_v[pl.ds(i * 16, 16)] = pv
        return acc + e
    with jax.named_scope("ph1_eloop"):
        acc_e = lax.fori_loop(0, NV, e_body, zero16)

    row_v[pl.ds(0, 16)] = acc_e
    pltpu.sync_copy(row_v, ps_hbm.at[c * 16 + s])

    with jax.named_scope("ph2_scat"):
        pltpu.async_copy(pval_v, t_hbm.at[key2_v], sem).wait()

    with jax.named_scope("ph2b_barrier"):
        plsc.subcore_barrier()

    with jax.named_scope("ph3_gath"):
        pltpu.async_copy(t_hbm.at[key2_v], t_v, sem).wait()

    def zero_body(k, _):
        s_local[pl.ds(k * 16, 16)] = zero16
        return 0
    with jax.named_scope("ph4_zero"):
        lax.fori_loop(0, SROWS // 16, zero_body, 0)

    def acc_body(i, _):
        kv = key2_v[pl.ds(i * 16, 16)]
        pv = pval_v[pl.ds(i * 16, 16)]
        tv = t_v[pl.ds(i * 16, 16)]
        m = (kv != kdummy) & (tv == pv)
        dv = dst_v[pl.ds(i * 16, 16)]
        sv = src_v[pl.ds(i * 16, 16)]
        w = e_v[pl.ds(i * 16, 16)] * plsc.load_gather(nodes_v, [dv])
        plsc.addupdate_scatter(s_local, [sv], w, mask=m)
        return 0
    with jax.named_scope("ph5_accloop"):
        lax.fori_loop(0, NV, acc_body, 0)

    with jax.named_scope("ph6_out"):
        pltpu.sync_copy(s_local, s32_hbm.at[c * 16 + s])


def _tc_body(s32_ref, ps_ref, w_ref, o_ref):
    denom = jnp.sum(ps_ref[:16, :])
    ssum = jnp.sum(s32_ref[...], axis=0, keepdims=True)
    ssum = ssum[:, :N_NODES] * (jnp.float32(1.0) / denom)
    out = lax.dot_general(ssum, w_ref[...], (((0,), (0,)), ((), ())),
                          preferred_element_type=jnp.float32)
    o_ref[0] = jnp.where(out > 0, out, out * jnp.float32(0.01))


@jax.jit
def kernel(x, src, dst, W_node, a):
    nodes = x[0, N_EDGES:]
    srcp = jnp.pad(src.astype(jnp.int32), (0, PPAD - N_PAIRS))
    dstp = jnp.pad(dst.astype(jnp.int32), (0, PPAD - N_PAIRS))
    wa = jnp.concatenate([W_node[0], a[:F_OUT, 0], a[F_OUT:, 0]])

    mesh = plsc.VectorSubcoreMesh(core_axis_name="c", subcore_axis_name="s",
                                  num_cores=2, num_subcores=16)
    sc = pl.kernel(
        _sc_body,
        mesh=mesh,
        compiler_params=pltpu.CompilerParams(needs_layout_passes=False),
        out_type=[
            jax.ShapeDtypeStruct((32, SROWS), jnp.float32),
            jax.ShapeDtypeStruct((32, 16), jnp.float32),
        ],
        scratch_types=[
            pltpu.HBM((TSIZE,), jnp.int32),
            pltpu.VMEM((N_NODES,), jnp.float32),
            pltpu.VMEM((CH,), jnp.int32),
            pltpu.VMEM((CH,), jnp.int32),
            pltpu.VMEM((CH,), jnp.int32),
            pltpu.VMEM((CH,), jnp.int32),
            pltpu.VMEM((CH,), jnp.float32),
            pltpu.VMEM((CH,), jnp.int32),
            pltpu.VMEM((SROWS,), jnp.float32),
            pltpu.VMEM((384,), jnp.float32),
            pltpu.VMEM((16,), jnp.float32),
            pltpu.SemaphoreType.DMA,
        ],
    )
    s32, ps = sc(nodes, srcp, dstp, wa)

    out = pl.pallas_call(
        _tc_body,
        out_shape=jax.ShapeDtypeStruct((1, N_NODES, F_OUT), jnp.float32),
    )(s32, ps, W_node)
    return out


# speedup vs baseline: 60.3463x; 59.7003x over previous
import jax
import jax.numpy as jnp
from jax import lax
from jax.experimental import pallas as pl
from jax.experimental.pallas import tpu as pltpu
from jax.experimental.pallas import tpu_sc as plsc

N_NODES = 10000
N_EDGES = 160000
N_PAIRS = 170000
F_OUT = 128

CH = 10752
PPAD = 16 * CH
NV = CH // 16
HALF = 50_000_000
LSTRIDE = 16
TROW = (HALF + 16384 + 8) * LSTRIDE
SROWS = 10240


def _keys_for_chunk(src_v, dst_v, key2_v, pval_v, c, s, per_pair):
    base = s * CH
    lanes = lax.iota(jnp.int32, 16)
    klo = c * HALF
    khi = klo + HALF

    def body(i, acc):
        sv = src_v[pl.ds(i * 16, 16)]
        dv = dst_v[pl.ds(i * 16, 16)]
        pv = base + i * 16 + lanes
        valid = pv < N_PAIRS
        key = sv * N_NODES + dv
        own = valid & (key >= klo) & (key < khi)
        kdummy = HALF + (pv & 16383)
        key2_v[pl.ds(i * 16, 16)] = (
            jnp.where(own, key - klo, kdummy) * LSTRIDE)
        pval_v[pl.ds(i * 16, 16)] = pv
        return per_pair(i, acc, sv, dv, pv, valid)

    zero16 = jnp.zeros((16,), jnp.float32)
    return lax.fori_loop(0, NV, body, zero16)


def _sc_scat_body(src_hbm, dst_hbm,
                  t0_hbm, t1_hbm,
                  src_v, dst_v, key2_v, pval_v, sem):
    c = lax.axis_index("c")
    s = lax.axis_index("s")
    base = s * CH
    pltpu.sync_copy(src_hbm.at[pl.ds(base, CH)], src_v)
    pltpu.sync_copy(dst_hbm.at[pl.ds(base, CH)], dst_v)

    def per_pair(i, acc, sv, dv, pv, valid):
        return acc
    _keys_for_chunk(src_v, dst_v, key2_v, pval_v, c, s, per_pair)

    @pl.when(c == 0)
    def _():
        pltpu.async_copy(pval_v, t0_hbm.at[key2_v], sem).wait()

    @pl.when(c == 1)
    def _():
        pltpu.async_copy(pval_v, t1_hbm.at[key2_v], sem).wait()


def _sc_acc_body(nodes_hbm, src_hbm, dst_hbm, wa_hbm, t0_hbm, t1_hbm,
                 s32_hbm, ps_hbm,
                 nodes_v, src_v, dst_v, key2_v, pval_v, e_v, t_v, s_local,
                 wa_v, row_v, sem):
    c = lax.axis_index("c")
    s = lax.axis_index("s")
    base = s * CH

    pltpu.sync_copy(nodes_hbm, nodes_v)
    pltpu.sync_copy(wa_hbm, wa_v)
    pltpu.sync_copy(src_hbm.at[pl.ds(base, CH)], src_v)
    pltpu.sync_copy(dst_hbm.at[pl.ds(base, CH)], dst_v)

    def dot_body(i, carry):
        a1v, a2v = carry
        w = wa_v[pl.ds(i * 16, 16)]
        return (a1v + w * wa_v[pl.ds(128 + i * 16, 16)],
                a2v + w * wa_v[pl.ds(256 + i * 16, 16)])
    zero16 = jnp.zeros((16,), jnp.float32)
    acc1, acc2 = lax.fori_loop(0, 8, dot_body, (zero16, zero16))
    c1 = jnp.sum(acc1, axis=0)
    c2 = jnp.sum(acc2, axis=0)

    def per_pair(i, acc, sv, dv, pv, valid):
        ns = plsc.load_gather(nodes_v, [sv])
        nd = plsc.load_gather(nodes_v, [dv])
        z = c1 * ns + c2 * nd
        z = jnp.maximum(z, z * jnp.float32(0.01))
        e = jnp.where(valid, jnp.exp(z), jnp.float32(0.0))
        e_v[pl.ds(i * 16, 16)] = e
        return acc + e
    acc_e = _keys_for_chunk(src_v, dst_v, key2_v, pval_v, c, s, per_pair)

    row_v[pl.ds(0, 16)] = acc_e
    pltpu.sync_copy(row_v, ps_hbm.at[c * 16 + s])

    @pl.when(c == 0)
    def _():
        pltpu.async_copy(t0_hbm.at[key2_v], t_v, sem).wait()

    @pl.when(c == 1)
    def _():
        pltpu.async_copy(t1_hbm.at[key2_v], t_v, sem).wait()

    def zero_body(k, _):
        s_local[pl.ds(k * 16, 16)] = zero16
        return 0
    lax.fori_loop(0, SROWS // 16, zero_body, 0)

    def acc_body(i, _):
        kv = key2_v[pl.ds(i * 16, 16)]
        pv = pval_v[pl.ds(i * 16, 16)]
        tv = t_v[pl.ds(i * 16, 16)]
        m = (kv < HALF * LSTRIDE) & (tv == pv)
        dv = dst_v[pl.ds(i * 16, 16)]
        sv = src_v[pl.ds(i * 16, 16)]
        w = e_v[pl.ds(i * 16, 16)] * plsc.load_gather(nodes_v, [dv])
        plsc.addupdate_scatter(s_local, [sv], w, mask=m)
        return 0
    lax.fori_loop(0, NV, acc_body, 0)

    pltpu.sync_copy(s_local, s32_hbm.at[c * 16 + s])


def _tc_body(s32_ref, ps_ref, w_ref, o_ref):
    denom = jnp.sum(ps_ref[:16, :])
    ssum = jnp.sum(s32_ref[...], axis=0, keepdims=True)
    ssum = ssum[:, :N_NODES] * (jnp.float32(1.0) / denom)
    out = lax.dot_general(ssum, w_ref[...], (((0,), (0,)), ((), ())),
                          preferred_element_type=jnp.float32)
    o_ref[0] = jnp.where(out > 0, out, out * jnp.float32(0.01))


@jax.jit
def kernel(x, src, dst, W_node, a):
    nodes = x[0, N_EDGES:]
    srcp = jnp.pad(src.astype(jnp.int32), (0, PPAD - N_PAIRS))
    dstp = jnp.pad(dst.astype(jnp.int32), (0, PPAD - N_PAIRS))
    wa = jnp.concatenate([W_node[0], a[:F_OUT, 0], a[F_OUT:, 0]])

    mesh = plsc.VectorSubcoreMesh(core_axis_name="c", subcore_axis_name="s",
                                  num_cores=2, num_subcores=16)
    params = pltpu.CompilerParams(needs_layout_passes=False)

    t0, t1 = pl.kernel(
        _sc_scat_body,
        mesh=mesh,
        compiler_params=params,
        out_type=[
            jax.ShapeDtypeStruct((TROW,), jnp.int32),
            jax.ShapeDtypeStruct((TROW,), jnp.int32),
        ],
        scratch_types=[
            pltpu.VMEM((CH,), jnp.int32),
            pltpu.VMEM((CH,), jnp.int32),
            pltpu.VMEM((CH,), jnp.int32),
            pltpu.VMEM((CH,), jnp.int32),
            pltpu.SemaphoreType.DMA,
        ],
    )(srcp, dstp)

    s32, ps = pl.kernel(
        _sc_acc_body,
        mesh=mesh,
        compiler_params=params,
        out_type=[
            jax.ShapeDtypeStruct((32, SROWS), jnp.float32),
            jax.ShapeDtypeStruct((32, 16), jnp.float32),
        ],
        scratch_types=[
            pltpu.VMEM((N_NODES,), jnp.float32),
            pltpu.VMEM((CH,), jnp.int32),
            pltpu.VMEM((CH,), jnp.int32),
            pltpu.VMEM((CH,), jnp.int32),
            pltpu.VMEM((CH,), jnp.int32),
            pltpu.VMEM((CH,), jnp.float32),
            pltpu.VMEM((CH,), jnp.int32),
            pltpu.VMEM((SROWS,), jnp.float32),
            pltpu.VMEM((384,), jnp.float32),
            pltpu.VMEM((16,), jnp.float32),
            pltpu.SemaphoreType.DMA,
        ],
    )(nodes, srcp, dstp, wa, t0, t1)

    out = pl.pallas_call(
        _tc_body,
        out_shape=jax.ShapeDtypeStruct((1, N_NODES, F_OUT), jnp.float32),
    )(s32, ps, W_node)
    return out
